# baseline (speedup 1.0000x reference)
import jax
import jax.numpy as jnp
from jax.experimental import pallas as pl
from jax.experimental.pallas import tpu as pltpu

TOPK = 8
BLOCK_M = 1024
KSPLIT = 2


def _router_body(scale_ref, x_ref, w_ref, w_out, i_out, acc):
    kid = pl.program_id(1)
    part = jax.lax.dot_general(
        w_ref[...], x_ref[...], (((1,), (1,)), ((), ())),
        preferred_element_type=jnp.float32)

    @pl.when(kid == 0)
    def _():
        acc[...] = part

    @pl.when(kid == KSPLIT - 1)
    def _():
        cur = (acc[...] + part) * scale_ref[0]
        n_exp = cur.shape[0]
        row = jax.lax.broadcasted_iota(jnp.int32, cur.shape, 0)
        vals, idxs = [], []
        for _ in range(TOPK):
            m = jnp.max(cur, axis=0, keepdims=True)
            idx = jnp.min(jnp.where(cur == m, row, n_exp), axis=0,
                          keepdims=True)
            vals.append(m)
            idxs.append(idx)
            cur = jnp.where(row == idx, -jnp.inf, cur)
        w = jnp.concatenate(vals, axis=0)
        e = jnp.exp(w - w[:1])
        w = e / jnp.sum(e, axis=0, keepdims=True)
        w_out[...] = w.T
        i_out[...] = jnp.concatenate(idxs, axis=0).T


@jax.jit
def kernel(x, W, router_scale):
    tokens, dim = x.shape
    n_exp = W.shape[0]
    kc = dim // KSPLIT
    grid = (tokens // BLOCK_M, KSPLIT)
    weights, indices = pl.pallas_call(
        _router_body,
        grid_spec=pltpu.PrefetchScalarGridSpec(
            num_scalar_prefetch=1,
            grid=grid,
            in_specs=[
                pl.BlockSpec((BLOCK_M, kc), lambda i, k, s: (i, k)),
                pl.BlockSpec((n_exp, kc), lambda i, k, s: (0, k)),
            ],
            out_specs=[
                pl.BlockSpec((BLOCK_M, TOPK), lambda i, k, s: (i, 0)),
                pl.BlockSpec((BLOCK_M, TOPK), lambda i, k, s: (i, 0)),
            ],
            scratch_shapes=[pltpu.VMEM((n_exp, BLOCK_M), jnp.float32)],
        ),
        out_shape=[
            jax.ShapeDtypeStruct((tokens, TOPK), jnp.float32),
            jax.ShapeDtypeStruct((tokens, TOPK), jnp.int32),
        ],
        compiler_params=pltpu.CompilerParams(
            dimension_semantics=("arbitrary", "arbitrary"),
        ),
    )(router_scale, x, W)
    return (weights, indices)


# speedup vs baseline: 1.1551x; 1.1551x over previous
import jax
import jax.numpy as jnp
from jax.experimental import pallas as pl
from jax.experimental.pallas import tpu as pltpu

TOPK = 8
BLOCK_M = 1024


def _router_body(scale_ref, x_ref, w_ref, w_out, i_out):
    lt = jax.lax.dot_general(
        w_ref[...], x_ref[...], (((1,), (1,)), ((), ())),
        preferred_element_type=jnp.float32)
    cur = lt * scale_ref[0]
    n_exp = cur.shape[0]
    row = jax.lax.broadcasted_iota(jnp.int32, cur.shape, 0)
    vals, idxs = [], []
    for _ in range(TOPK):
        m = jnp.max(cur, axis=0, keepdims=True)
        idx = jnp.min(jnp.where(cur == m, row, n_exp), axis=0, keepdims=True)
        vals.append(m)
        idxs.append(idx)
        cur = jnp.where(row == idx, -jnp.inf, cur)
    w = jnp.concatenate(vals, axis=0)
    e = jnp.exp(w - w[:1])
    w = e / jnp.sum(e, axis=0, keepdims=True)
    w_out[...] = w.T
    i_out[...] = jnp.concatenate(idxs, axis=0).T


@jax.jit
def kernel(x, W, router_scale):
    tokens, dim = x.shape
    n_exp = W.shape[0]
    grid = (tokens // BLOCK_M,)
    weights, indices = pl.pallas_call(
        _router_body,
        grid_spec=pltpu.PrefetchScalarGridSpec(
            num_scalar_prefetch=1,
            grid=grid,
            in_specs=[
                pl.BlockSpec((BLOCK_M, dim), lambda i, s: (i, 0)),
                pl.BlockSpec((n_exp, dim), lambda i, s: (0, 0)),
            ],
            out_specs=[
                pl.BlockSpec((BLOCK_M, TOPK), lambda i, s: (i, 0)),
                pl.BlockSpec((BLOCK_M, TOPK), lambda i, s: (i, 0)),
            ],
        ),
        out_shape=[
            jax.ShapeDtypeStruct((tokens, TOPK), jnp.float32),
            jax.ShapeDtypeStruct((tokens, TOPK), jnp.int32),
        ],
        compiler_params=pltpu.CompilerParams(
            dimension_semantics=("arbitrary",),
            vmem_limit_bytes=128 * 1024 * 1024,
        ),
    )(router_scale, x, W)
    return (weights, indices)
